# baseline (speedup 1.0000x reference)
import jax
import jax.numpy as jnp
from jax import lax
from jax.experimental import pallas as pl
from jax.experimental.pallas import tpu as pltpu
from jax.experimental.pallas import tpu_sc as plsc

_B = 16384
_D = 64
_V = 1000000
_NC = 2
_NS = 16
_NW = _NC * _NS
_BPW = _B // _NW
_CHUNK = 128
_NCHUNK = _BPW // _CHUNK
_GROUPS = _CHUNK // 16

_NBIN = _V // 128
_TAIL = _NBIN * 128


def _conv_body(eut, eit, tlu, tli, us_hbm, is_hbm,
               slab_u0, slab_u1, slab_i0, slab_i1,
               out_u0, out_u1, out_i0, out_i1,
               sem_u0, sem_u1, sem_i0, sem_i1, sem_o0, sem_o1):
    wid = lax.axis_index("s") * _NC + lax.axis_index("c")
    extra = jnp.minimum(wid, 2)
    start = wid * 244 + extra * 2
    nb = 244 + jnp.where(wid < 2, 2, 0)
    last = start + nb - 1

    slabs = ((slab_u0, slab_i0, out_u0, out_i0, sem_u0, sem_i0, sem_o0),
             (slab_u1, slab_i1, out_u1, out_i1, sem_u1, sem_i1, sem_o1))

    iota = lax.iota(jnp.int32, 16)
    kb = [(jb % 4) * 2 + iota // 8 for jb in range(8)]
    ki = [iota % 8 for _ in range(8)]

    def issue_in(b, slot):
        su, si, _, _, mu, mi, _ = slabs[slot]
        pltpu.async_copy(eut.at[:, :, pl.ds(b * 128, 128)], su, mu)
        pltpu.async_copy(eit.at[:, :, pl.ds(b * 128, 128)], si, mi)

    def shuffle(slab, outb):
        def prow(p, carry):
            la = jnp.full((16,), 0, jnp.int32) + 2 * p
            lb = la + 1
            for jb in range(8):
                lane = la if jb < 4 else lb
                g = plsc.load_gather(slab, [kb[jb], ki[jb], lane])
                outb[p, pl.ds(jb * 16, 16)] = g
            return carry

        lax.fori_loop(0, 64, prow, 0)

    issue_in(start, 0)
    issue_in(start + 1, 1)

    def body(t2, carry):
        b0 = start + 2 * t2
        for slot in (0, 1):
            su, si, ou, oi, mu, mi, mo = slabs[slot]
            b = b0 + slot
            pltpu.make_async_copy(eut.at[:, :, pl.ds(0, 128)], su, mu).wait()
            pltpu.make_async_copy(eit.at[:, :, pl.ds(0, 128)], si, mi).wait()
            @pl.when(t2 > 0)
            def _():
                pltpu.make_async_copy(us_hbm.at[pl.ds(0, 64)], ou, mo).wait()
                pltpu.make_async_copy(is_hbm.at[pl.ds(0, 64)], oi, mo).wait()

            shuffle(su, ou)
            shuffle(si, oi)
            pltpu.async_copy(ou, us_hbm.at[pl.ds(b * 64, 64)], mo)
            pltpu.async_copy(oi, is_hbm.at[pl.ds(b * 64, 64)], mo)
            bn = jnp.minimum(b + 2, last)
            issue_in(bn, slot)
        return carry

    lax.fori_loop(0, nb // 2, body, 0)

    for slot in (0, 1):
        su, si, ou, oi, mu, mi, mo = slabs[slot]
        pltpu.make_async_copy(eut.at[:, :, pl.ds(0, 128)], su, mu).wait()
        pltpu.make_async_copy(eit.at[:, :, pl.ds(0, 128)], si, mi).wait()
        pltpu.make_async_copy(us_hbm.at[pl.ds(0, 64)], ou, mo).wait()
        pltpu.make_async_copy(is_hbm.at[pl.ds(0, 64)], oi, mo).wait()

    @pl.when(wid == 0)
    def _():
        pltpu.sync_copy(tlu, out_u0.at[pl.ds(0, 32)])
        pltpu.sync_copy(out_u0.at[pl.ds(0, 32)], us_hbm.at[pl.ds(_TAIL // 2, 32)])
        pltpu.sync_copy(tli, out_i0.at[pl.ds(0, 32)])
        pltpu.sync_copy(out_i0.at[pl.ds(0, 32)], is_hbm.at[pl.ds(_TAIL // 2, 32)])


def _bpr_body(upk_hbm, ipk_hbm, jpk_hbm, uhf_hbm, ihf_hbm, jhf_hbm,
              eu_hbm, ei_hbm, oi_hbm, oj_hbm,
              upk, ipk, jpk, uhf, ihf, jhf,
              ua, ub, via, vib, vja, vjb,
              oi_v, oj_v, sem_a, sem_b, sem_i):
    wid = lax.axis_index("s") * _NC + lax.axis_index("c")
    base = wid * _BPW

    cps0 = [pltpu.async_copy(src.at[wid], dst, sem_i)
            for src, dst in ((upk_hbm, upk), (ipk_hbm, ipk), (jpk_hbm, jpk),
                             (uhf_hbm, uhf), (ihf_hbm, ihf), (jhf_hbm, jhf))]
    for cp in cps0:
        cp.wait()

    rowbufs = ((ua, via, vja, sem_a), (ub, vib, vjb, sem_b))

    def issue(c, slot):
        ubuf, vibuf, vjbuf, sem = rowbufs[slot]
        return (pltpu.async_copy(eu_hbm.at[upk.at[c]], ubuf, sem),
                pltpu.async_copy(ei_hbm.at[ipk.at[c]], vibuf, sem),
                pltpu.async_copy(ei_hbm.at[jpk.at[c]], vjbuf, sem))

    iota = lax.iota(jnp.int32, 16)

    def compute(c, slot):
        ubuf, vibuf, vjbuf, _ = rowbufs[slot]

        def group(g, carry):
            rows = g * 16 + iota
            goff = g * 16
            uoff = uhf[c, pl.ds(goff, 16)]
            ioff = ihf[c, pl.ds(goff, 16)]
            joff = jhf[c, pl.ds(goff, 16)]
            acc_i = jnp.zeros((16,), jnp.float32)
            acc_j = jnp.zeros((16,), jnp.float32)
            for k in range(_D):
                u = plsc.load_gather(ubuf, [rows, uoff + k])
                vi = plsc.load_gather(vibuf, [rows, ioff + k])
                vj = plsc.load_gather(vjbuf, [rows, joff + k])
                acc_i = acc_i + u * vi
                acc_j = acc_j + u * vj
            off = c * _CHUNK + goff
            oi_v[pl.ds(off, 16)] = acc_i
            oj_v[pl.ds(off, 16)] = acc_j
            return carry

        lax.fori_loop(0, _GROUPS, group, 0)

    cps = issue(0, 0)
    for c in range(_NCHUNK):
        slot = c % 2
        for cp in cps:
            cp.wait()
        if c + 1 < _NCHUNK:
            nxt = issue(c + 1, 1 - slot)
        compute(c, slot)
        if c + 1 < _NCHUNK:
            cps = nxt

    o0 = pltpu.async_copy(oi_v, oi_hbm.at[pl.ds(base, _BPW)], sem_i)
    o1 = pltpu.async_copy(oj_v, oj_hbm.at[pl.ds(base, _BPW)], sem_i)
    o0.wait()
    o1.wait()


def kernel(user, item_i, item_j, embed_user, embed_item):
    f32 = jnp.float32
    i32 = jnp.int32
    mesh = plsc.VectorSubcoreMesh(core_axis_name="c", subcore_axis_name="s")
    params = pltpu.CompilerParams(needs_layout_passes=False)

    conv = pl.kernel(
        _conv_body,
        out_type=(jax.ShapeDtypeStruct((_V // 2, 128), f32),
                  jax.ShapeDtypeStruct((_V // 2, 128), f32)),
        mesh=mesh,
        compiler_params=params,
        scratch_types=(
            [pltpu.VMEM((8, 8, 128), f32) for _ in range(4)]
            + [pltpu.VMEM((64, 128), f32) for _ in range(4)]
            + [pltpu.SemaphoreType.DMA for _ in range(6)]
        ),
    )

    run = pl.kernel(
        _bpr_body,
        out_type=(jax.ShapeDtypeStruct((_B,), f32),
                  jax.ShapeDtypeStruct((_B,), f32)),
        mesh=mesh,
        compiler_params=params,
        scratch_types=(
            [pltpu.VMEM((_NCHUNK, _CHUNK), i32) for _ in range(6)]
            + [pltpu.VMEM((_CHUNK, 2 * _D), f32) for _ in range(6)]
            + [pltpu.VMEM((_BPW,), f32) for _ in range(2)]
            + [pltpu.SemaphoreType.DMA for _ in range(3)]
        ),
    )

    eut = embed_user.T.reshape(8, 8, _V)
    eit = embed_item.T.reshape(8, 8, _V)
    tlu = embed_user[_TAIL:].reshape(32, 128)
    tli = embed_item[_TAIL:].reshape(32, 128)
    eu2, ei2 = conv(eut, eit, tlu, tli)

    shp = (_NW, _NCHUNK, _CHUNK)
    upk = (user >> 1).reshape(shp)
    ipk = (item_i >> 1).reshape(shp)
    jpk = (item_j >> 1).reshape(shp)
    uhf = ((user & 1) * _D).reshape(shp)
    ihf = ((item_i & 1) * _D).reshape(shp)
    jhf = ((item_j & 1) * _D).reshape(shp)
    return run(upk, ipk, jpk, uhf, ihf, jhf, eu2, ei2)


# speedup vs baseline: 2.6717x; 2.6717x over previous
import jax
import jax.numpy as jnp
from jax import lax
from jax.experimental import pallas as pl
from jax.experimental.pallas import tpu as pltpu
from jax.experimental.pallas import tpu_sc as plsc

_B = 16384
_D = 64
_NC = 2
_NS = 16
_NW = _NC * _NS
_BPW = _B // _NW
_CHUNK = 128
_NCHUNK = _BPW // _CHUNK
_GROUPS = _CHUNK // 16


def _bpr_body(upk_hbm, ipk_hbm, jpk_hbm, uhf_hbm, ihf_hbm, jhf_hbm,
              eu_hbm, ei_hbm, oi_hbm, oj_hbm,
              upk, ipk, jpk, uhf, ihf, jhf,
              ua, ub, via, vib, vja, vjb,
              oi_v, oj_v, sem_a, sem_b, sem_i):
    wid = lax.axis_index("s") * _NC + lax.axis_index("c")
    base = wid * _BPW

    cps0 = [pltpu.async_copy(src.at[wid], dst, sem_i)
            for src, dst in ((upk_hbm, upk), (ipk_hbm, ipk), (jpk_hbm, jpk),
                             (uhf_hbm, uhf), (ihf_hbm, ihf), (jhf_hbm, jhf))]
    for cp in cps0:
        cp.wait()

    rowbufs = ((ua, via, vja, sem_a), (ub, vib, vjb, sem_b))

    def issue(c, slot):
        ubuf, vibuf, vjbuf, sem = rowbufs[slot]
        return (pltpu.async_copy(eu_hbm.at[upk.at[c]], ubuf, sem),
                pltpu.async_copy(ei_hbm.at[ipk.at[c]], vibuf, sem),
                pltpu.async_copy(ei_hbm.at[jpk.at[c]], vjbuf, sem))

    iota = lax.iota(jnp.int32, 16)

    def compute(c, slot):
        ubuf, vibuf, vjbuf, _ = rowbufs[slot]

        def group(g, carry):
            rows = g * 16 + iota
            goff = g * 16
            uoff = uhf[c, pl.ds(goff, 16)]
            ioff = ihf[c, pl.ds(goff, 16)]
            joff = jhf[c, pl.ds(goff, 16)]
            acc_i = jnp.zeros((16,), jnp.float32)
            acc_j = jnp.zeros((16,), jnp.float32)
            for k in range(_D):
                u = plsc.load_gather(ubuf, [rows, uoff + k])
                vi = plsc.load_gather(vibuf, [rows, ioff + k])
                vj = plsc.load_gather(vjbuf, [rows, joff + k])
                acc_i = acc_i + u * vi
                acc_j = acc_j + u * vj
            off = c * _CHUNK + goff
            oi_v[pl.ds(off, 16)] = acc_i
            oj_v[pl.ds(off, 16)] = acc_j
            return carry

        lax.fori_loop(0, _GROUPS, group, 0)

    cps = issue(0, 0)
    for c in range(_NCHUNK):
        slot = c % 2
        for cp in cps:
            cp.wait()
        if c + 1 < _NCHUNK:
            nxt = issue(c + 1, 1 - slot)
        compute(c, slot)
        if c + 1 < _NCHUNK:
            cps = nxt

    o0 = pltpu.async_copy(oi_v, oi_hbm.at[pl.ds(base, _BPW)], sem_i)
    o1 = pltpu.async_copy(oj_v, oj_hbm.at[pl.ds(base, _BPW)], sem_i)
    o0.wait()
    o1.wait()


def kernel(user, item_i, item_j, embed_user, embed_item):
    f32 = jnp.float32
    mesh = plsc.VectorSubcoreMesh(core_axis_name="c", subcore_axis_name="s")
    run = pl.kernel(
        _bpr_body,
        out_type=(jax.ShapeDtypeStruct((_B,), f32),
                  jax.ShapeDtypeStruct((_B,), f32)),
        mesh=mesh,
        compiler_params=pltpu.CompilerParams(needs_layout_passes=False),
        scratch_types=[
            pltpu.VMEM((_NCHUNK, _CHUNK), jnp.int32),
            pltpu.VMEM((_NCHUNK, _CHUNK), jnp.int32),
            pltpu.VMEM((_NCHUNK, _CHUNK), jnp.int32),
            pltpu.VMEM((_NCHUNK, _CHUNK), jnp.int32),
            pltpu.VMEM((_NCHUNK, _CHUNK), jnp.int32),
            pltpu.VMEM((_NCHUNK, _CHUNK), jnp.int32),
            pltpu.VMEM((_CHUNK, 2 * _D), f32),
            pltpu.VMEM((_CHUNK, 2 * _D), f32),
            pltpu.VMEM((_CHUNK, 2 * _D), f32),
            pltpu.VMEM((_CHUNK, 2 * _D), f32),
            pltpu.VMEM((_CHUNK, 2 * _D), f32),
            pltpu.VMEM((_CHUNK, 2 * _D), f32),
            pltpu.VMEM((_BPW,), f32),
            pltpu.VMEM((_BPW,), f32),
            pltpu.SemaphoreType.DMA,
            pltpu.SemaphoreType.DMA,
            pltpu.SemaphoreType.DMA,
        ],
    )
    eu2 = embed_user.reshape(embed_user.shape[0] // 2, 2 * _D)
    ei2 = embed_item.reshape(embed_item.shape[0] // 2, 2 * _D)
    shp = (_NW, _NCHUNK, _CHUNK)
    upk = (user >> 1).reshape(shp)
    ipk = (item_i >> 1).reshape(shp)
    jpk = (item_j >> 1).reshape(shp)
    uhf = ((user & 1) * _D).reshape(shp)
    ihf = ((item_i & 1) * _D).reshape(shp)
    jhf = ((item_j & 1) * _D).reshape(shp)
    return run(upk, ipk, jpk, uhf, ihf, jhf, eu2, ei2)
